# baseline (speedup 1.0000x reference)
import functools
import math

import jax
import jax.numpy as jnp
from jax import lax
from jax.experimental import pallas as pl
from jax.experimental.pallas import tpu as pltpu
from jax.experimental.pallas import tpu_sc as plsc

VOCAB = 1000000
EMB = 64
NC = 2
NS = 16
NW = NC * NS
LANE = 128
SCALE = math.sqrt(EMB)


def _emb_kernel(n_seq, table_hbm, idx_hbm, out_hbm,
                idx_v, in0, in1, ob0, ob1, gsem0, gsem1, osem0, osem1):
    w = lax.axis_index("s") * NC + lax.axis_index("c")

    pltpu.sync_copy(idx_hbm.at[w], idx_v)

    pltpu.async_copy(table_hbm.at[idx_v.at[0]], in0, gsem0)
    pltpu.async_copy(table_hbm.at[idx_v.at[1]], in1, gsem1)

    def do_block(s, buf_in, buf_ob, gsem, osem):
        pltpu.make_async_copy(table_hbm.at[idx_v.at[s]], buf_in, gsem).wait()

        @pl.when(s >= 2)
        def _():
            pltpu.make_async_copy(buf_ob, out_hbm.at[s - 2, :, w], osem).wait()

        def col_group(k):
            rows = lax.iota(jnp.int32, 16) + k * 16
            for r in range(EMB // 8):
                for i in range(8):
                    col = jnp.full((16,), 8 * r + i, jnp.int32)
                    vals = plsc.load_gather(buf_in, [rows, col])
                    buf_ob[r, i, pl.ds(k * 16, 16)] = vals * SCALE

        pl.loop(0, LANE // 16)(col_group)

        pltpu.async_copy(buf_ob, out_hbm.at[s, :, w], osem)

        @pl.when(s + 2 < n_seq)
        def _():
            pltpu.async_copy(table_hbm.at[idx_v.at[s + 2]], buf_in, gsem)

    def body(j):
        do_block(j, in0, ob0, gsem0, osem0)
        do_block(j + 1, in1, ob1, gsem1, osem1)

    pl.loop(0, n_seq, step=2)(body)

    pltpu.make_async_copy(ob0, out_hbm.at[n_seq - 2, :, w], osem0).wait()
    pltpu.make_async_copy(ob1, out_hbm.at[n_seq - 1, :, w], osem1).wait()


def kernel(tokens, table):
    n_batch, n_seq = tokens.shape
    assert n_batch % (NW * LANE) == 0 or n_batch == NW * LANE
    assert n_batch == NW * LANE and n_seq % 2 == 0

    idx = jnp.transpose(
        jnp.reshape(tokens.astype(jnp.int32), (NW, LANE, n_seq)), (0, 2, 1))

    mesh = plsc.VectorSubcoreMesh(
        core_axis_name="c", subcore_axis_name="s",
        num_cores=NC, num_subcores=NS)

    run = functools.partial(
        pl.kernel,
        out_type=jax.ShapeDtypeStruct((n_seq, EMB // 8, NW, 8, LANE),
                                      jnp.float32),
        mesh=mesh,
        compiler_params=pltpu.CompilerParams(
            use_tc_tiling_on_sc=False, needs_layout_passes=False),
        scratch_types=[
            pltpu.VMEM((n_seq, LANE), jnp.int32),
            pltpu.VMEM((LANE, EMB), jnp.float32),
            pltpu.VMEM((LANE, EMB), jnp.float32),
            pltpu.VMEM((EMB // 8, 8, LANE), jnp.float32),
            pltpu.VMEM((EMB // 8, 8, LANE), jnp.float32),
            pltpu.SemaphoreType.DMA,
            pltpu.SemaphoreType.DMA,
            pltpu.SemaphoreType.DMA,
            pltpu.SemaphoreType.DMA,
        ],
    )(functools.partial(_emb_kernel, n_seq))

    out5 = run(table, idx)
    out = jnp.transpose(out5, (2, 4, 0, 1, 3)).reshape(
        NW * LANE, n_seq, EMB)
    return out


# speedup vs baseline: 1.1336x; 1.1336x over previous
import functools
import math

import jax
import jax.numpy as jnp
from jax import lax
from jax.experimental import pallas as pl
from jax.experimental.pallas import tpu as pltpu
from jax.experimental.pallas import tpu_sc as plsc

VOCAB = 1000000
EMB = 64
NC = 2
NS = 16
NW = NC * NS
LANE = 128
SCALE = math.sqrt(EMB)


def _emb_kernel(n_seq, table_hbm, idx_hbm, out_hbm,
                idx_v, in0, in1, ob0, ob1, gsem0, gsem1, osem0, osem1):
    w = lax.axis_index("s") * NC + lax.axis_index("c")

    pltpu.sync_copy(idx_hbm.at[w], idx_v)

    pltpu.async_copy(table_hbm.at[idx_v.at[0]], in0, gsem0)
    pltpu.async_copy(table_hbm.at[idx_v.at[1]], in1, gsem1)

    iota = lax.iota(jnp.int32, 16)
    r_vecs = [(c * 16 + iota) >> 3 for c in range(EMB // 16)]
    m_vecs = [((c * 16 + iota) & 7) * LANE for c in range(EMB // 16)]

    def do_block(s, buf_in, buf_ob, gsem, osem):
        pltpu.make_async_copy(table_hbm.at[idx_v.at[s]], buf_in, gsem).wait()

        @pl.when(s >= 2)
        def _():
            pltpu.make_async_copy(buf_ob, out_hbm.at[s - 2, :, w], osem).wait()

        def token_pair(j):
            for dj in range(2):
                jb = jnp.full((16,), j + dj, jnp.int32)
                for c in range(EMB // 16):
                    vals = buf_in[j + dj, pl.ds(c * 16, 16)] * SCALE
                    plsc.store_scatter(buf_ob, [r_vecs[c], m_vecs[c] + jb],
                                       vals)

        pl.loop(0, LANE, step=2)(token_pair)

        pltpu.async_copy(buf_ob, out_hbm.at[s, :, w], osem)

        @pl.when(s + 2 < n_seq)
        def _():
            pltpu.async_copy(table_hbm.at[idx_v.at[s + 2]], buf_in, gsem)

    def body(j):
        do_block(j, in0, ob0, gsem0, osem0)
        do_block(j + 1, in1, ob1, gsem1, osem1)

    pl.loop(0, n_seq, step=2)(body)

    pltpu.make_async_copy(ob0, out_hbm.at[n_seq - 2, :, w], osem0).wait()
    pltpu.make_async_copy(ob1, out_hbm.at[n_seq - 1, :, w], osem1).wait()


def kernel(tokens, table):
    n_batch, n_seq = tokens.shape
    assert n_batch == NW * LANE and n_seq % 2 == 0

    idx = jnp.transpose(
        jnp.reshape(tokens.astype(jnp.int32), (NW, LANE, n_seq)), (0, 2, 1))

    mesh = plsc.VectorSubcoreMesh(
        core_axis_name="c", subcore_axis_name="s",
        num_cores=NC, num_subcores=NS)

    run = functools.partial(
        pl.kernel,
        out_type=jax.ShapeDtypeStruct((n_seq, EMB // 8, NW, 8 * LANE),
                                      jnp.float32),
        mesh=mesh,
        compiler_params=pltpu.CompilerParams(
            use_tc_tiling_on_sc=False, needs_layout_passes=False),
        scratch_types=[
            pltpu.VMEM((n_seq, LANE), jnp.int32),
            pltpu.VMEM((LANE, EMB), jnp.float32),
            pltpu.VMEM((LANE, EMB), jnp.float32),
            pltpu.VMEM((EMB // 8, 8 * LANE), jnp.float32),
            pltpu.VMEM((EMB // 8, 8 * LANE), jnp.float32),
            pltpu.SemaphoreType.DMA,
            pltpu.SemaphoreType.DMA,
            pltpu.SemaphoreType.DMA,
            pltpu.SemaphoreType.DMA,
        ],
    )(functools.partial(_emb_kernel, n_seq))

    out4 = run(table, idx)
    out5 = jnp.reshape(out4, (n_seq, EMB // 8, NW, 8, LANE))
    out = jnp.transpose(out5, (2, 4, 0, 1, 3)).reshape(
        NW * LANE, n_seq, EMB)
    return out


# speedup vs baseline: 1.4945x; 1.3184x over previous
import functools
import math

import jax
import jax.numpy as jnp
from jax import lax
from jax.experimental import pallas as pl
from jax.experimental.pallas import tpu as pltpu
from jax.experimental.pallas import tpu_sc as plsc

VOCAB = 1000000
EMB = 64
NC = 2
NS = 16
NW = NC * NS
LANE = 128
SCALE = math.sqrt(EMB)


def _emb_kernel(n_seq, table_hbm, idx_hbm, out_hbm,
                idx_v, in0, in1, ob0, ob1, gsem0, gsem1, osem0, osem1):
    w = lax.axis_index("s") * NC + lax.axis_index("c")

    pltpu.sync_copy(idx_hbm.at[w], idx_v)

    pltpu.async_copy(table_hbm.at[idx_v.at[0]], in0, gsem0)
    pltpu.async_copy(table_hbm.at[idx_v.at[1]], in1, gsem1)

    iota = lax.iota(jnp.int32, 16)
    r_vecs = [(c * 16 + iota) >> 3 for c in range(EMB // 16)]
    m_vecs = [((c * 16 + iota) & 7) * LANE for c in range(EMB // 16)]

    def do_block(s, buf_in, buf_ob, gsem, osem):
        pltpu.make_async_copy(table_hbm.at[idx_v.at[s]], buf_in, gsem).wait()

        @pl.when(s >= 2)
        def _():
            pltpu.make_async_copy(buf_ob, out_hbm.at[s - 2, :, w], osem).wait()

        @plsc.parallel_loop(0, LANE, step=2, unroll=4)
        def token_pair(j):
            for dj in range(2):
                jb = jnp.full((16,), j + dj, jnp.int32)
                for c in range(EMB // 16):
                    vals = buf_in[j + dj, pl.ds(c * 16, 16)] * SCALE
                    plsc.store_scatter(buf_ob, [r_vecs[c], m_vecs[c] + jb],
                                       vals)

        pltpu.async_copy(buf_ob, out_hbm.at[s, :, w], osem)

        @pl.when(s + 2 < n_seq)
        def _():
            pltpu.async_copy(table_hbm.at[idx_v.at[s + 2]],
                             buf_in.at[:, pl.ds(0, EMB)], gsem)

    def body(j):
        do_block(j, in0, ob0, gsem0, osem0)
        do_block(j + 1, in1, ob1, gsem1, osem1)

    pl.loop(0, n_seq, step=2)(body)

    pltpu.make_async_copy(ob0, out_hbm.at[n_seq - 2, :, w], osem0).wait()
    pltpu.make_async_copy(ob1, out_hbm.at[n_seq - 1, :, w], osem1).wait()


def kernel(tokens, table):
    n_batch, n_seq = tokens.shape
    assert n_batch == NW * LANE and n_seq % 2 == 0

    idx = jnp.transpose(
        jnp.reshape(tokens.astype(jnp.int32), (NW, LANE, n_seq)), (0, 2, 1))

    mesh = plsc.VectorSubcoreMesh(
        core_axis_name="c", subcore_axis_name="s",
        num_cores=NC, num_subcores=NS)

    run = functools.partial(
        pl.kernel,
        out_type=jax.ShapeDtypeStruct((n_seq, EMB // 8, NW, 8 * LANE),
                                      jnp.float32),
        mesh=mesh,
        compiler_params=pltpu.CompilerParams(
            use_tc_tiling_on_sc=False, needs_layout_passes=False),
        scratch_types=[
            pltpu.VMEM((n_seq, LANE), jnp.int32),
            pltpu.VMEM((LANE, EMB), jnp.float32),
            pltpu.VMEM((LANE, EMB), jnp.float32),
            pltpu.VMEM((EMB // 8, 8 * LANE), jnp.float32),
            pltpu.VMEM((EMB // 8, 8 * LANE), jnp.float32),
            pltpu.SemaphoreType.DMA,
            pltpu.SemaphoreType.DMA,
            pltpu.SemaphoreType.DMA,
            pltpu.SemaphoreType.DMA,
        ],
    )(functools.partial(_emb_kernel, n_seq))

    out4 = run(table, idx)
    out5 = jnp.reshape(out4, (n_seq, EMB // 8, NW, 8, LANE))
    out = jnp.transpose(out5, (2, 4, 0, 1, 3)).reshape(
        NW * LANE, n_seq, EMB)
    return out


# speedup vs baseline: 2.1399x; 1.4319x over previous
import functools
import math

import jax
import jax.numpy as jnp
from jax import lax
from jax.experimental import pallas as pl
from jax.experimental.pallas import tpu as pltpu
from jax.experimental.pallas import tpu_sc as plsc

VOCAB = 1000000
EMB = 64
NC = 2
NS = 16
NW = NC * NS
LANE = 128
SCALE = math.sqrt(EMB)


def _emb_kernel(n_seq, table_hbm, idx_hbm, out_hbm,
                idx_v, in0, in1, ob0, ob1, gsem0, gsem1, osem0, osem1):
    w = lax.axis_index("s") * NC + lax.axis_index("c")

    pltpu.sync_copy(idx_hbm.at[:, w], idx_v)

    def idx_row(s):
        return idx_v.at[s >> 3, s & 7]

    pltpu.async_copy(table_hbm.at[idx_row(0)], in0, gsem0)
    pltpu.async_copy(table_hbm.at[idx_row(1)], in1, gsem1)

    iota = lax.iota(jnp.int32, 16)
    rows_j0 = [j0 + iota for j0 in range(0, LANE, 16)]

    def do_block(s, buf_in, buf_ob, gsem, osem):
        pltpu.make_async_copy(table_hbm.at[idx_row(s)], buf_in, gsem).wait()

        @pl.when(s >= 2)
        def _():
            pltpu.make_async_copy(buf_ob, out_hbm.at[s - 2, :, w], osem).wait()

        @plsc.parallel_loop(0, 16, unroll=2)
        def diag(d):
            perm = (iota + d) & 15
            ph = perm >> 3
            pm = (perm & 7) * LANE
            cols = [perm + 16 * c for c in range(EMB // 16)]
            rvec = [ph + 2 * c for c in range(EMB // 16)]
            for g in range(LANE // 16):
                mv = pm + rows_j0[g]
                for c in range(EMB // 16):
                    vals = plsc.load_gather(buf_in, [rows_j0[g], cols[c]])
                    plsc.store_scatter(buf_ob, [rvec[c], mv], vals * SCALE)

        pltpu.async_copy(buf_ob, out_hbm.at[s, :, w], osem)

        @pl.when(s + 2 < n_seq)
        def _():
            pltpu.async_copy(table_hbm.at[idx_row(s + 2)], buf_in, gsem)

    def body(j):
        do_block(j, in0, ob0, gsem0, osem0)
        do_block(j + 1, in1, ob1, gsem1, osem1)

    pl.loop(0, n_seq, step=2)(body)

    pltpu.make_async_copy(ob0, out_hbm.at[n_seq - 2, :, w], osem0).wait()
    pltpu.make_async_copy(ob1, out_hbm.at[n_seq - 1, :, w], osem1).wait()


def kernel(tokens, table):
    n_batch, n_seq = tokens.shape
    assert n_batch == NW * LANE and n_seq % 2 == 0

    idx = jnp.transpose(
        jnp.reshape(tokens.astype(jnp.int32).T, (n_seq // 8, 8, NW, LANE)),
        (0, 2, 1, 3))

    mesh = plsc.VectorSubcoreMesh(
        core_axis_name="c", subcore_axis_name="s",
        num_cores=NC, num_subcores=NS)

    run = functools.partial(
        pl.kernel,
        out_type=jax.ShapeDtypeStruct((n_seq, EMB // 8, NW, 8 * LANE),
                                      jnp.float32),
        mesh=mesh,
        compiler_params=pltpu.CompilerParams(
            use_tc_tiling_on_sc=False, needs_layout_passes=False),
        scratch_types=[
            pltpu.VMEM((n_seq // 8, 8, LANE), jnp.int32),
            pltpu.VMEM((LANE, EMB), jnp.float32),
            pltpu.VMEM((LANE, EMB), jnp.float32),
            pltpu.VMEM((EMB // 8, 8 * LANE), jnp.float32),
            pltpu.VMEM((EMB // 8, 8 * LANE), jnp.float32),
            pltpu.SemaphoreType.DMA,
            pltpu.SemaphoreType.DMA,
            pltpu.SemaphoreType.DMA,
            pltpu.SemaphoreType.DMA,
        ],
    )(functools.partial(_emb_kernel, n_seq))

    out4 = run(table, idx)
    out5 = jnp.reshape(out4, (n_seq, EMB // 8, NW, 8, LANE))
    out = jnp.transpose(out5, (2, 4, 0, 1, 3)).reshape(
        NW * LANE, n_seq, EMB)
    return out
